# baseline (speedup 1.0000x reference)
import functools

import numpy as np
import jax
import jax.numpy as jnp
from jax import lax
from jax.experimental import pallas as pl
from jax.experimental.pallas import tpu as pltpu
from jax.experimental.pallas import tpu_sc as plsc

_CHUNK = 128
_BLK = 2048


def _dft_weights(D):
    n = np.arange(D)
    ang = 2.0 * np.pi * np.outer(n, n) / D
    return np.cos(ang).astype(np.float32), (-np.sin(ang)).astype(np.float32)


@functools.cache
def _make_gather(V, D, B):
    info = plsc.get_sparse_core_info()
    NC, NS = info.num_cores, info.num_subcores
    NW = NC * NS
    n_ch = B // (NW * _CHUNK)
    mesh = plsc.VectorSubcoreMesh(core_axis_name="c", subcore_axis_name="s")

    @functools.partial(
        pl.kernel,
        mesh=mesh,
        out_type=jax.ShapeDtypeStruct((B, D), jnp.float32),
        compiler_params=pltpu.CompilerParams(use_tc_tiling_on_sc=False),
        scratch_types=[
            pltpu.VMEM((n_ch, _CHUNK), jnp.int32),
            pltpu.VMEM((_CHUNK, D), jnp.float32),
            pltpu.SemaphoreType.DMA,
        ],
    )
    def gather(table_hbm, idx_hbm, out_hbm, idx_v, rows_v, sem):
        wid = lax.axis_index("s") * NC + lax.axis_index("c")
        pltpu.sync_copy(idx_hbm.at[pl.ds(wid * n_ch, n_ch)], idx_v)

        def body(i, carry):
            pltpu.async_copy(table_hbm.at[idx_v.at[i]], rows_v, sem).wait()
            row0 = (wid * n_ch + i) * _CHUNK
            pltpu.sync_copy(rows_v, out_hbm.at[pl.ds(row0, _CHUNK)])
            return carry

        lax.fori_loop(0, n_ch, body, 0)

    return gather


@functools.cache
def _make_dft(B, D):
    def body(x_ref, wc_ref, ws_ref, re_ref, im_ref):
        x = x_ref[...]
        re_ref[...] = jnp.dot(x, wc_ref[...], preferred_element_type=jnp.float32)
        im_ref[...] = jnp.dot(x, ws_ref[...], preferred_element_type=jnp.float32)

    return pl.pallas_call(
        body,
        grid=(B // _BLK,),
        in_specs=[
            pl.BlockSpec((_BLK, D), lambda i: (i, 0)),
            pl.BlockSpec((D, D), lambda i: (0, 0)),
            pl.BlockSpec((D, D), lambda i: (0, 0)),
        ],
        out_specs=[
            pl.BlockSpec((_BLK, D), lambda i: (i, 0)),
            pl.BlockSpec((_BLK, D), lambda i: (i, 0)),
        ],
        out_shape=[
            jax.ShapeDtypeStruct((B, D), jnp.float32),
            jax.ShapeDtypeStruct((B, D), jnp.float32),
        ],
    )


def kernel(emb_weight, toks):
    Bt, H = toks.shape
    V, D = emb_weight.shape
    B = Bt * H
    idx = toks.reshape(B // _CHUNK, _CHUNK).astype(jnp.int32)
    x = _make_gather(V, D, B)(emb_weight, idx)
    wc, ws = _dft_weights(D)
    re, im = _make_dft(B, D)(x, jnp.asarray(wc), jnp.asarray(ws))
    return lax.complex(re, im).reshape(Bt, H, D)


# speedup vs baseline: 1.1577x; 1.1577x over previous
import functools

import numpy as np
import jax
import jax.numpy as jnp
from jax import lax
from jax.experimental import pallas as pl
from jax.experimental.pallas import tpu as pltpu
from jax.experimental.pallas import tpu_sc as plsc

_CHUNK = 128


def _dft_weights(D):
    n = np.arange(D)
    ang = 2.0 * np.pi * np.outer(n, n) / D
    return np.cos(ang).astype(np.float32), (-np.sin(ang)).astype(np.float32)


@functools.cache
def _make_gather(V, D, B):
    info = plsc.get_sparse_core_info()
    NC, NS = info.num_cores, info.num_subcores
    NW = NC * NS
    n_ch = B // (NW * _CHUNK)
    mesh = plsc.VectorSubcoreMesh(core_axis_name="c", subcore_axis_name="s")

    @functools.partial(
        pl.kernel,
        mesh=mesh,
        out_type=jax.ShapeDtypeStruct((B, D), jnp.float32),
        compiler_params=pltpu.CompilerParams(use_tc_tiling_on_sc=False),
        scratch_types=[
            pltpu.VMEM((n_ch, _CHUNK), jnp.int32),
            pltpu.VMEM((_CHUNK, D), jnp.float32),
            pltpu.SemaphoreType.DMA,
        ],
    )
    def gather(table_hbm, idx_hbm, out_hbm, idx_v, rows_v, sem):
        wid = lax.axis_index("s") * NC + lax.axis_index("c")
        pltpu.sync_copy(idx_hbm.at[pl.ds(wid * n_ch, n_ch)], idx_v)

        def body(i, carry):
            pltpu.async_copy(table_hbm.at[idx_v.at[i]], rows_v, sem).wait()
            row0 = (wid * n_ch + i) * _CHUNK
            pltpu.sync_copy(rows_v, out_hbm.at[pl.ds(row0, _CHUNK)])
            return carry

        lax.fori_loop(0, n_ch, body, 0)

    return gather


@functools.cache
def _make_dft_h(B, H, D):
    dn = (((1,), (1,)), ((), ()))

    def body(x_ref, wc_ref, ws_ref, re_ref, im_ref):
        x = x_ref[0]
        re_ref[0] = lax.dot_general(wc_ref[...], x, dn,
                                    preferred_element_type=jnp.float32)
        im_ref[0] = lax.dot_general(ws_ref[...], x, dn,
                                    preferred_element_type=jnp.float32)

    return pl.pallas_call(
        body,
        grid=(H,),
        in_specs=[
            pl.BlockSpec((1, B, D), lambda h: (h, 0, 0)),
            pl.BlockSpec((D, D), lambda h: (0, 0)),
            pl.BlockSpec((D, D), lambda h: (0, 0)),
        ],
        out_specs=[
            pl.BlockSpec((1, D, B), lambda h: (h, 0, 0)),
            pl.BlockSpec((1, D, B), lambda h: (h, 0, 0)),
        ],
        out_shape=[
            jax.ShapeDtypeStruct((H, D, B), jnp.float32),
            jax.ShapeDtypeStruct((H, D, B), jnp.float32),
        ],
    )


def kernel(emb_weight, toks):
    Bt, H = toks.shape
    V, D = emb_weight.shape
    B = Bt * H
    idxT = jnp.transpose(toks).astype(jnp.int32).reshape(B // _CHUNK, _CHUNK)
    xT = _make_gather(V, D, B)(emb_weight, idxT)
    wc, ws = _dft_weights(D)
    r, i = _make_dft_h(Bt, H, D)(xT.reshape(H, Bt, D),
                                 jnp.asarray(wc.T.copy()),
                                 jnp.asarray(ws.T.copy()))
    return lax.complex(r.transpose(2, 0, 1), i.transpose(2, 0, 1))


# speedup vs baseline: 1.2018x; 1.0381x over previous
import functools

import numpy as np
import jax
import jax.numpy as jnp
from jax import lax
from jax.experimental import pallas as pl
from jax.experimental.pallas import tpu as pltpu
from jax.experimental.pallas import tpu_sc as plsc

_CHUNK = 128


def _dft_weights(D):
    n = np.arange(D)
    ang = 2.0 * np.pi * np.outer(n, n) / D
    return np.cos(ang).astype(np.float32), (-np.sin(ang)).astype(np.float32)


@functools.cache
def _make_gather(V, D, B):
    info = plsc.get_sparse_core_info()
    NC, NS = info.num_cores, info.num_subcores
    NW = NC * NS
    n_ch = B // (NW * _CHUNK)
    mesh = plsc.VectorSubcoreMesh(core_axis_name="c", subcore_axis_name="s")

    @functools.partial(
        pl.kernel,
        mesh=mesh,
        out_type=jax.ShapeDtypeStruct((B, D), jnp.float32),
        compiler_params=pltpu.CompilerParams(use_tc_tiling_on_sc=False),
        scratch_types=[
            pltpu.VMEM((n_ch, _CHUNK), jnp.int32),
            pltpu.VMEM((_CHUNK, D), jnp.float32),
            pltpu.SemaphoreType.DMA,
        ],
    )
    def gather(table_hbm, idx_hbm, out_hbm, idx_v, rows_v, sem):
        wid = lax.axis_index("s") * NC + lax.axis_index("c")
        pltpu.sync_copy(idx_hbm.at[pl.ds(wid * n_ch, n_ch)], idx_v)

        def body(i, carry):
            pltpu.async_copy(table_hbm.at[idx_v.at[i]], rows_v, sem).wait()
            row0 = (wid * n_ch + i) * _CHUNK
            pltpu.sync_copy(rows_v, out_hbm.at[pl.ds(row0, _CHUNK)])
            return carry

        lax.fori_loop(0, n_ch, body, 0)

    return gather


@functools.cache
def _make_dft_h(B, H, D):
    dn = (((1,), (1,)), ((), ()))
    P = 2 * D

    def body(x_ref, wc_ref, ws_ref, re_ref, im_ref):
        x2 = x_ref[0]
        xe = x2[:, :D]
        xo = x2[:, D:]
        wc = wc_ref[...]
        ws = ws_ref[...]
        re_ref[0] = jnp.concatenate(
            [lax.dot_general(wc, xe, dn, preferred_element_type=jnp.float32),
             lax.dot_general(wc, xo, dn, preferred_element_type=jnp.float32)],
            axis=1)
        im_ref[0] = jnp.concatenate(
            [lax.dot_general(ws, xe, dn, preferred_element_type=jnp.float32),
             lax.dot_general(ws, xo, dn, preferred_element_type=jnp.float32)],
            axis=1)

    return pl.pallas_call(
        body,
        grid=(H,),
        in_specs=[
            pl.BlockSpec((1, B * D // P, P), lambda h: (h, 0, 0)),
            pl.BlockSpec((D, D), lambda h: (0, 0)),
            pl.BlockSpec((D, D), lambda h: (0, 0)),
        ],
        out_specs=[
            pl.BlockSpec((1, D, B), lambda h: (h, 0, 0)),
            pl.BlockSpec((1, D, B), lambda h: (h, 0, 0)),
        ],
        out_shape=[
            jax.ShapeDtypeStruct((H, D, B), jnp.float32),
            jax.ShapeDtypeStruct((H, D, B), jnp.float32),
        ],
    )


def kernel(emb_weight, toks):
    Bt, H = toks.shape
    V, D = emb_weight.shape
    B = Bt * H
    idxT = jnp.transpose(
        toks.astype(jnp.int32).reshape(2, Bt // 2, H), (2, 1, 0)
    ).reshape(B // _CHUNK, _CHUNK)
    xT = _make_gather(V, D, B)(emb_weight, idxT)
    wc, ws = _dft_weights(D)
    r, i = _make_dft_h(Bt, H, D)(xT.reshape(H, Bt * D // (2 * D), 2 * D),
                                 jnp.asarray(wc.T.copy()),
                                 jnp.asarray(ws.T.copy()))
    return lax.complex(r.transpose(2, 0, 1), i.transpose(2, 0, 1))


# speedup vs baseline: 1.2434x; 1.0347x over previous
import functools

import numpy as np
import jax
import jax.numpy as jnp
from jax import lax
from jax.experimental import pallas as pl
from jax.experimental.pallas import tpu as pltpu
from jax.experimental.pallas import tpu_sc as plsc

_CHUNK = 128


def _dft_weights(D):
    n = np.arange(D)
    ang = 2.0 * np.pi * np.outer(n, n) / D
    return np.cos(ang).astype(np.float32), (-np.sin(ang)).astype(np.float32)


@functools.cache
def _make_gather(V, D, B):
    info = plsc.get_sparse_core_info()
    NC, NS = info.num_cores, info.num_subcores
    NW = NC * NS
    n_ch = B // (NW * _CHUNK)
    mesh = plsc.VectorSubcoreMesh(core_axis_name="c", subcore_axis_name="s")

    @functools.partial(
        pl.kernel,
        mesh=mesh,
        out_type=jax.ShapeDtypeStruct((B // 2, 2 * D), jnp.float32),
        compiler_params=pltpu.CompilerParams(use_tc_tiling_on_sc=False),
        scratch_types=[
            pltpu.VMEM((n_ch, _CHUNK), jnp.int32),
            pltpu.VMEM((_CHUNK, D), jnp.float32),
            pltpu.SemaphoreType.DMA,
        ],
    )
    def gather(table_hbm, idx_hbm, out_hbm, idx_v, rows_v, sem):
        wid = lax.axis_index("s") * NC + lax.axis_index("c")
        pltpu.sync_copy(idx_hbm.at[pl.ds(wid * n_ch, n_ch)], idx_v)

        def body(i, carry):
            pltpu.async_copy(table_hbm.at[idx_v.at[i]], rows_v, sem).wait()
            row0 = (wid * n_ch + i) * (_CHUNK // 2)
            half = _CHUNK // 2
            pltpu.sync_copy(rows_v.at[pl.ds(0, half)],
                            out_hbm.at[pl.ds(row0, half), pl.ds(0, D)])
            pltpu.sync_copy(rows_v.at[pl.ds(half, half)],
                            out_hbm.at[pl.ds(row0, half), pl.ds(D, D)])
            return carry

        lax.fori_loop(0, n_ch, body, 0)

    return gather


@functools.cache
def _make_dft_h(B, H, D):
    dn = (((1,), (1,)), ((), ()))
    P = 2 * D

    def body(x_ref, wc_ref, ws_ref, re_ref, im_ref):
        x2 = x_ref[0]
        xe = x2[:, :D]
        xo = x2[:, D:]
        wc = wc_ref[...]
        ws = ws_ref[...]
        re_ref[0] = jnp.concatenate(
            [lax.dot_general(wc, xe, dn, preferred_element_type=jnp.float32),
             lax.dot_general(wc, xo, dn, preferred_element_type=jnp.float32)],
            axis=1)
        im_ref[0] = jnp.concatenate(
            [lax.dot_general(ws, xe, dn, preferred_element_type=jnp.float32),
             lax.dot_general(ws, xo, dn, preferred_element_type=jnp.float32)],
            axis=1)

    return pl.pallas_call(
        body,
        grid=(H,),
        in_specs=[
            pl.BlockSpec((1, B * D // P, P), lambda h: (h, 0, 0)),
            pl.BlockSpec((D, D), lambda h: (0, 0)),
            pl.BlockSpec((D, D), lambda h: (0, 0)),
        ],
        out_specs=[
            pl.BlockSpec((1, D, B), lambda h: (h, 0, 0)),
            pl.BlockSpec((1, D, B), lambda h: (h, 0, 0)),
        ],
        out_shape=[
            jax.ShapeDtypeStruct((H, D, B), jnp.float32),
            jax.ShapeDtypeStruct((H, D, B), jnp.float32),
        ],
    )


def kernel(emb_weight, toks):
    Bt, H = toks.shape
    V, D = emb_weight.shape
    B = Bt * H
    G = _CHUNK // 2
    idxT = jnp.transpose(
        toks.astype(jnp.int32).reshape(2, (Bt // 2) // G, G, H), (3, 1, 0, 2)
    ).reshape(B // _CHUNK, _CHUNK)
    xT = _make_gather(V, D, B)(emb_weight, idxT)
    wc, ws = _dft_weights(D)
    r, i = _make_dft_h(Bt, H, D)(xT.reshape(H, Bt * D // (2 * D), 2 * D),
                                 jnp.asarray(wc.T.copy()),
                                 jnp.asarray(ws.T.copy()))
    return lax.complex(r.transpose(2, 0, 1), i.transpose(2, 0, 1))


# speedup vs baseline: 1.2863x; 1.0345x over previous
import functools

import numpy as np
import jax
import jax.numpy as jnp
from jax import lax
from jax.experimental import pallas as pl
from jax.experimental.pallas import tpu as pltpu
from jax.experimental.pallas import tpu_sc as plsc

_CHUNK = 128


def _dft_weights(D):
    n = np.arange(D)
    ang = 2.0 * np.pi * np.outer(n, n) / D
    return np.cos(ang).astype(np.float32), (-np.sin(ang)).astype(np.float32)


@functools.cache
def _make_gather(V, D, B):
    info = plsc.get_sparse_core_info()
    NC, NS = info.num_cores, info.num_subcores
    NW = NC * NS
    n_ch = B // (NW * _CHUNK)
    mesh = plsc.VectorSubcoreMesh(core_axis_name="c", subcore_axis_name="s")

    GRP = 4
    n_grp = n_ch // GRP
    half = _CHUNK // 2

    @functools.partial(
        pl.kernel,
        mesh=mesh,
        out_type=jax.ShapeDtypeStruct((B // 2, 2 * D), jnp.float32),
        compiler_params=pltpu.CompilerParams(use_tc_tiling_on_sc=False),
        scratch_types=[
            pltpu.VMEM((n_ch, _CHUNK), jnp.int32),
            pltpu.VMEM((2, GRP, _CHUNK, D), jnp.float32),
            pltpu.SemaphoreType.DMA,
            pltpu.SemaphoreType.DMA,
        ],
    )
    def gather(table_hbm, idx_hbm, out_hbm, idx_v, rows_v, sem_g, sem_w):
        wid = lax.axis_index("s") * NC + lax.axis_index("c")
        pltpu.sync_copy(idx_hbm.at[pl.ds(wid * n_ch, n_ch)], idx_v)

        def gather_grp(g, slot):
            for b in range(GRP):
                pltpu.async_copy(table_hbm.at[idx_v.at[g * GRP + b]],
                                 rows_v.at[slot, b], sem_g)

        def out_descs(g, slot):
            ds = []
            for b in range(GRP):
                row0 = (wid * n_ch + g * GRP + b) * half
                ds.append(pltpu.make_async_copy(
                    rows_v.at[slot, b, pl.ds(0, half)],
                    out_hbm.at[pl.ds(row0, half), pl.ds(0, D)], sem_w))
                ds.append(pltpu.make_async_copy(
                    rows_v.at[slot, b, pl.ds(half, half)],
                    out_hbm.at[pl.ds(row0, half), pl.ds(D, D)], sem_w))
            return ds

        def wait_gather_grp(g, slot):
            for b in range(GRP):
                pltpu.make_async_copy(table_hbm.at[idx_v.at[g * GRP + b]],
                                      rows_v.at[slot, b], sem_g).wait()

        gather_grp(0, 0)

        def step(g, slot):
            other = 1 - slot
            wait_gather_grp(g, slot)

            @pl.when(g >= 1)
            def _():
                for d in out_descs(g - 1, other):
                    d.wait()

            @pl.when(g + 1 < n_grp)
            def _():
                gather_grp(g + 1, other)

            for d in out_descs(g, slot):
                d.start()

        def body(gg, carry):
            step(2 * gg, 0)
            step(2 * gg + 1, 1)
            return carry

        lax.fori_loop(0, n_grp // 2, body, 0)
        for d in out_descs(n_grp - 1, (n_grp - 1) % 2):
            d.wait()

    return gather


@functools.cache
def _make_dft_h(B, H, D):
    dn = (((1,), (1,)), ((), ()))
    P = 2 * D

    def body(x_ref, wc_ref, ws_ref, re_ref, im_ref):
        x2 = x_ref[0]
        xe = x2[:, :D]
        xo = x2[:, D:]
        wc = wc_ref[...]
        ws = ws_ref[...]
        re_ref[0] = jnp.concatenate(
            [lax.dot_general(wc, xe, dn, preferred_element_type=jnp.float32),
             lax.dot_general(wc, xo, dn, preferred_element_type=jnp.float32)],
            axis=1)
        im_ref[0] = jnp.concatenate(
            [lax.dot_general(ws, xe, dn, preferred_element_type=jnp.float32),
             lax.dot_general(ws, xo, dn, preferred_element_type=jnp.float32)],
            axis=1)

    return pl.pallas_call(
        body,
        grid=(H,),
        in_specs=[
            pl.BlockSpec((1, B * D // P, P), lambda h: (h, 0, 0)),
            pl.BlockSpec((D, D), lambda h: (0, 0)),
            pl.BlockSpec((D, D), lambda h: (0, 0)),
        ],
        out_specs=[
            pl.BlockSpec((1, D, B), lambda h: (h, 0, 0)),
            pl.BlockSpec((1, D, B), lambda h: (h, 0, 0)),
        ],
        out_shape=[
            jax.ShapeDtypeStruct((H, D, B), jnp.float32),
            jax.ShapeDtypeStruct((H, D, B), jnp.float32),
        ],
    )


def kernel(emb_weight, toks):
    Bt, H = toks.shape
    V, D = emb_weight.shape
    B = Bt * H
    G = _CHUNK // 2
    idxT = jnp.transpose(
        toks.astype(jnp.int32).reshape(2, (Bt // 2) // G, G, H), (3, 1, 0, 2)
    ).reshape(B // _CHUNK, _CHUNK)
    xT = _make_gather(V, D, B)(emb_weight, idxT)
    wc, ws = _dft_weights(D)
    r, i = _make_dft_h(Bt, H, D)(xT.reshape(H, Bt * D // (2 * D), 2 * D),
                                 jnp.asarray(wc.T.copy()),
                                 jnp.asarray(ws.T.copy()))
    return lax.complex(r.transpose(2, 0, 1), i.transpose(2, 0, 1))
